# Initial kernel scaffold; baseline (speedup 1.0000x reference)
import jax
import jax.numpy as jnp
from jax.experimental import pallas as pl


def kernel(x, w_ij, idx_i, idx_j, W1, b1, W2, b2):
    raise NotImplementedError("write your pallas kernel here")


# speedup vs baseline: 2.4602x; 2.4602x over previous
import functools

import jax
import jax.numpy as jnp
from jax import lax
from jax.experimental import pallas as pl
from jax.experimental.pallas import tpu as pltpu
from jax.experimental.pallas import tpu_sc as plsc

N_NODES = 10000
N_PAIRS = 320000
F = 128


_PRE_BLK = 1000


def _premul_body(x_ref, w1a_ref, w1b_ref, xa_ref, xb_ref):
    x = x_ref[...]
    xa_ref[...] = jnp.dot(x, w1a_ref[...], preferred_element_type=jnp.float32)
    xb_ref[...] = jnp.dot(x, w1b_ref[...], preferred_element_type=jnp.float32)


def _premul(x, w1a, w1b):
    grid = (N_NODES // _PRE_BLK,)
    return pl.pallas_call(
        _premul_body,
        grid=grid,
        in_specs=[
            pl.BlockSpec((_PRE_BLK, F), lambda i: (i, 0)),
            pl.BlockSpec((F, F), lambda i: (0, 0)),
            pl.BlockSpec((F, F), lambda i: (0, 0)),
        ],
        out_specs=[
            pl.BlockSpec((_PRE_BLK, F), lambda i: (i, 0)),
            pl.BlockSpec((_PRE_BLK, F), lambda i: (i, 0)),
        ],
        out_shape=[
            jax.ShapeDtypeStruct((N_NODES, F), jnp.float32),
            jax.ShapeDtypeStruct((N_NODES, F), jnp.float32),
        ],
    )(x, w1a, w1b)


_CHUNK = 128
_NCHUNKS = N_PAIRS // _CHUNK
_NW = 32


def _gather_add_body(xa_hbm, xb_hbm, idxi_hbm, idxj_hbm, g_hbm,
                     idxa_v, idxb_v, bufa_v, bufb_v, sema, semb):
    wid = lax.axis_index("s") * 2 + lax.axis_index("c")
    rem = _NCHUNKS % _NW
    cnt = _NCHUNKS // _NW + jnp.where(wid < rem, 1, 0)

    def chunk_body(t, carry):
        base = (wid + t * _NW) * _CHUNK
        pltpu.sync_copy(idxi_hbm.at[pl.ds(base, _CHUNK)], idxa_v)
        pltpu.sync_copy(idxj_hbm.at[pl.ds(base, _CHUNK)], idxb_v)
        cpa = pltpu.async_copy(xa_hbm.at[idxa_v], bufa_v, sema)
        cpb = pltpu.async_copy(xb_hbm.at[idxb_v], bufb_v, semb)
        cpa.wait()
        cpb.wait()

        def add_row(r, c2):
            for k in range(F // 16):
                sl = pl.ds(k * 16, 16)
                bufa_v[r, sl] = bufa_v[r, sl] + bufb_v[r, sl]
            return c2

        lax.fori_loop(0, _CHUNK, add_row, 0)
        pltpu.sync_copy(bufa_v, g_hbm.at[pl.ds(base, _CHUNK)])
        return carry

    lax.fori_loop(0, cnt, chunk_body, 0)


@functools.cache
def _make_gather_add():
    return functools.partial(
        pl.kernel,
        out_type=jax.ShapeDtypeStruct((N_PAIRS, F), jnp.float32),
        mesh=plsc.VectorSubcoreMesh(core_axis_name="c", subcore_axis_name="s"),
        scratch_types=[
            pltpu.VMEM((_CHUNK,), jnp.int32),
            pltpu.VMEM((_CHUNK,), jnp.int32),
            pltpu.VMEM((_CHUNK, F), jnp.float32),
            pltpu.VMEM((_CHUNK, F), jnp.float32),
            pltpu.SemaphoreType.DMA,
            pltpu.SemaphoreType.DMA,
        ],
    )(_gather_add_body)


_MLP_BLK = 1280
_MLP_NB = N_PAIRS // _MLP_BLK


def _mlp_body(w_ref, g_ref, w1c_ref, b1_ref, w2_ref, b2_ref, out_ref):
    c = jnp.dot(w_ref[...], w1c_ref[...], preferred_element_type=jnp.float32)
    y = c + g_ref[...] + b1_ref[...]
    h = y * jax.nn.sigmoid(y)
    out_ref[0, 0, :] = jnp.sum(h * w2_ref[...], axis=1) + b2_ref[0]


def _mlp(w_ij, g, w1c, b1_row, w2_row, b2):
    out = pl.pallas_call(
        _mlp_body,
        grid=(_MLP_NB,),
        in_specs=[
            pl.BlockSpec((_MLP_BLK, F), lambda i: (i, 0)),
            pl.BlockSpec((_MLP_BLK, F), lambda i: (i, 0)),
            pl.BlockSpec((F, F), lambda i: (0, 0)),
            pl.BlockSpec((1, F), lambda i: (0, 0)),
            pl.BlockSpec((1, F), lambda i: (0, 0)),
            pl.BlockSpec(memory_space=pltpu.SMEM),
        ],
        out_specs=pl.BlockSpec((1, 1, _MLP_BLK), lambda i: (i, 0, 0)),
        out_shape=jax.ShapeDtypeStruct((_MLP_NB, 1, _MLP_BLK), jnp.float32),
    )(w_ij, g, w1c, b1_row, w2_row, b2)
    return out.reshape(N_PAIRS)


def kernel(x, w_ij, idx_i, idx_j, W1, b1, W2, b2):
    w1a = W1[:F]
    w1b = W1[F:2 * F]
    w1c = W1[2 * F:]
    xa, xb = _premul(x, w1a, w1b)
    g = _make_gather_add()(xa, xb, idx_i.astype(jnp.int32), idx_j.astype(jnp.int32))
    return _mlp(w_ij, g, w1c, b1.reshape(1, F), W2.reshape(1, F), b2)


# speedup vs baseline: 2.9020x; 1.1796x over previous
import functools

import jax
import jax.numpy as jnp
from jax import lax
from jax.experimental import pallas as pl
from jax.experimental.pallas import tpu as pltpu
from jax.experimental.pallas import tpu_sc as plsc

N_NODES = 10000
N_PAIRS = 320000
F = 128


_PRE_BLK = 1000


def _premul_body(x_ref, w1a_ref, w1b_ref, xa_ref, xb_ref):
    x = x_ref[...]
    xa_ref[...] = jnp.dot(x, w1a_ref[...], preferred_element_type=jnp.float32)
    xb_ref[...] = jnp.dot(x, w1b_ref[...], preferred_element_type=jnp.float32)


def _premul(x, w1a, w1b):
    grid = (N_NODES // _PRE_BLK,)
    return pl.pallas_call(
        _premul_body,
        grid=grid,
        in_specs=[
            pl.BlockSpec((_PRE_BLK, F), lambda i: (i, 0)),
            pl.BlockSpec((F, F), lambda i: (0, 0)),
            pl.BlockSpec((F, F), lambda i: (0, 0)),
        ],
        out_specs=[
            pl.BlockSpec((_PRE_BLK, F), lambda i: (i, 0)),
            pl.BlockSpec((_PRE_BLK, F), lambda i: (i, 0)),
        ],
        out_shape=[
            jax.ShapeDtypeStruct((N_NODES, F), jnp.float32),
            jax.ShapeDtypeStruct((N_NODES, F), jnp.float32),
        ],
    )(x, w1a, w1b)


_CHUNK = 128
_NCHUNKS = N_PAIRS // _CHUNK
_NW = 32


def _gather_add_body(xa_hbm, xb_hbm, idxi_hbm, idxj_hbm, g_hbm,
                     idxa0, idxb0, idxa1, idxb1,
                     bufa0, bufb0, bufa1, bufb1,
                     sa0, sb0, sa1, sb1):
    wid = lax.axis_index("s") * 2 + lax.axis_index("c")
    rem = _NCHUNKS % _NW
    cnt = _NCHUNKS // _NW + jnp.where(wid < rem, 1, 0)

    def fire(t, idxa, idxb, bufa, bufb, sa, sb):
        base = (wid + t * _NW) * _CHUNK
        pltpu.sync_copy(idxi_hbm.at[pl.ds(base, _CHUNK)], idxa)
        pltpu.sync_copy(idxj_hbm.at[pl.ds(base, _CHUNK)], idxb)
        cpa = pltpu.async_copy(xa_hbm.at[idxa], bufa, sa)
        cpb = pltpu.async_copy(xb_hbm.at[idxb], bufb, sb)
        return cpa, cpb

    def add_store(t, bufa, bufb):
        def add_row(r, c2):
            for k in range(F // 16):
                sl = pl.ds(k * 16, 16)
                bufa[r, sl] = bufa[r, sl] + bufb[r, sl]
            return c2

        lax.fori_loop(0, _CHUNK, add_row, 0)
        base = (wid + t * _NW) * _CHUNK
        pltpu.sync_copy(bufa, g_hbm.at[pl.ds(base, _CHUNK)])

    def pair_body(s, carry):
        t0 = 2 * s
        t1 = 2 * s + 1
        cp0 = fire(t0, idxa0, idxb0, bufa0, bufb0, sa0, sb0)
        cp1 = fire(t1, idxa1, idxb1, bufa1, bufb1, sa1, sb1)
        cp0[0].wait()
        cp0[1].wait()
        add_store(t0, bufa0, bufb0)
        cp1[0].wait()
        cp1[1].wait()
        add_store(t1, bufa1, bufb1)
        return carry

    lax.fori_loop(0, cnt // 2, pair_body, 0)

    @pl.when(cnt % 2 == 1)
    def _tail():
        t = cnt - 1
        cp = fire(t, idxa0, idxb0, bufa0, bufb0, sa0, sb0)
        cp[0].wait()
        cp[1].wait()
        add_store(t, bufa0, bufb0)


@functools.cache
def _make_gather_add():
    return functools.partial(
        pl.kernel,
        out_type=jax.ShapeDtypeStruct((N_PAIRS, F), jnp.float32),
        mesh=plsc.VectorSubcoreMesh(core_axis_name="c", subcore_axis_name="s"),
        scratch_types=[
            pltpu.VMEM((_CHUNK,), jnp.int32),
            pltpu.VMEM((_CHUNK,), jnp.int32),
            pltpu.VMEM((_CHUNK,), jnp.int32),
            pltpu.VMEM((_CHUNK,), jnp.int32),
            pltpu.VMEM((_CHUNK, F), jnp.float32),
            pltpu.VMEM((_CHUNK, F), jnp.float32),
            pltpu.VMEM((_CHUNK, F), jnp.float32),
            pltpu.VMEM((_CHUNK, F), jnp.float32),
            pltpu.SemaphoreType.DMA,
            pltpu.SemaphoreType.DMA,
            pltpu.SemaphoreType.DMA,
            pltpu.SemaphoreType.DMA,
        ],
    )(_gather_add_body)


_MLP_BLK = 1280
_MLP_NB = N_PAIRS // _MLP_BLK


def _mlp_body(w_ref, g_ref, w1c_ref, b1_ref, w2_ref, b2_ref, out_ref):
    c = jnp.dot(w_ref[...].astype(jnp.bfloat16), w1c_ref[...].astype(jnp.bfloat16),
                preferred_element_type=jnp.float32)
    y = c + g_ref[...] + b1_ref[...]
    h = y * jax.nn.sigmoid(y)
    out_ref[0, 0, :] = jnp.sum(h * w2_ref[...], axis=1) + b2_ref[0]


def _mlp(w_ij, g, w1c, b1_row, w2_row, b2):
    out = pl.pallas_call(
        _mlp_body,
        grid=(_MLP_NB,),
        in_specs=[
            pl.BlockSpec((_MLP_BLK, F), lambda i: (i, 0)),
            pl.BlockSpec((_MLP_BLK, F), lambda i: (i, 0)),
            pl.BlockSpec((F, F), lambda i: (0, 0)),
            pl.BlockSpec((1, F), lambda i: (0, 0)),
            pl.BlockSpec((1, F), lambda i: (0, 0)),
            pl.BlockSpec(memory_space=pltpu.SMEM),
        ],
        out_specs=pl.BlockSpec((1, 1, _MLP_BLK), lambda i: (i, 0, 0)),
        out_shape=jax.ShapeDtypeStruct((_MLP_NB, 1, _MLP_BLK), jnp.float32),
    )(w_ij, g, w1c, b1_row, w2_row, b2)
    return out.reshape(N_PAIRS)


def kernel(x, w_ij, idx_i, idx_j, W1, b1, W2, b2):
    w1a = W1[:F]
    w1b = W1[F:2 * F]
    w1c = W1[2 * F:]
    xa, xb = _premul(x, w1a, w1b)
    g = _make_gather_add()(xa, xb, idx_i.astype(jnp.int32), idx_j.astype(jnp.int32))
    return _mlp(w_ij, g, w1c, b1.reshape(1, F), W2.reshape(1, F), b2)


# speedup vs baseline: 3.2183x; 1.1090x over previous
import functools

import jax
import jax.numpy as jnp
from jax import lax
from jax.experimental import pallas as pl
from jax.experimental.pallas import tpu as pltpu
from jax.experimental.pallas import tpu_sc as plsc

N_NODES = 10000
N_PAIRS = 320000
F = 128


_PRE_BLK = 1000


def _premul_body(x_ref, w1a_ref, w1b_ref, xa_ref, xb_ref):
    x = x_ref[...]
    xa_ref[...] = jnp.dot(x, w1a_ref[...], preferred_element_type=jnp.float32)
    xb_ref[...] = jnp.dot(x, w1b_ref[...], preferred_element_type=jnp.float32)


def _premul(x, w1a, w1b):
    grid = (N_NODES // _PRE_BLK,)
    return pl.pallas_call(
        _premul_body,
        grid=grid,
        in_specs=[
            pl.BlockSpec((_PRE_BLK, F), lambda i: (i, 0)),
            pl.BlockSpec((F, F), lambda i: (0, 0)),
            pl.BlockSpec((F, F), lambda i: (0, 0)),
        ],
        out_specs=[
            pl.BlockSpec((_PRE_BLK, F), lambda i: (i, 0)),
            pl.BlockSpec((_PRE_BLK, F), lambda i: (i, 0)),
        ],
        out_shape=[
            jax.ShapeDtypeStruct((N_NODES, F), jnp.float32),
            jax.ShapeDtypeStruct((N_NODES, F), jnp.float32),
        ],
    )(x, w1a, w1b)


_CHUNK = 128
_NCHUNKS = N_PAIRS // _CHUNK
_NW = 32


def _gather_add_body(xa_hbm, xb_hbm, idxi_hbm, idxj_hbm, g_hbm,
                     idxa0, idxb0, idxa1, idxb1,
                     bufa0, bufb0, bufa1, bufb1,
                     sa0, sb0, sa1, sb1):
    wid = lax.axis_index("s") * 2 + lax.axis_index("c")
    rem = _NCHUNKS % _NW
    cnt = _NCHUNKS // _NW + jnp.where(wid < rem, 1, 0)

    def fire(t, idxa, idxb, bufa, bufb, sa, sb):
        base = (wid + t * _NW) * _CHUNK
        pltpu.sync_copy(idxi_hbm.at[pl.ds(base, _CHUNK)], idxa)
        pltpu.sync_copy(idxj_hbm.at[pl.ds(base, _CHUNK)], idxb)
        cpa = pltpu.async_copy(xa_hbm.at[idxa], bufa, sa)
        cpb = pltpu.async_copy(xb_hbm.at[idxb], bufb, sb)
        return cpa, cpb

    def add_store(t, bufa, bufb):
        def add_row(r, c2):
            for k in range(F // 16):
                sl = pl.ds(k * 16, 16)
                bufa[r, sl] = bufa[r, sl] + bufb[r, sl]
            return c2

        lax.fori_loop(0, _CHUNK, add_row, 0)
        base = (wid + t * _NW) * _CHUNK
        pltpu.sync_copy(bufa, g_hbm.at[pl.ds(base, _CHUNK)])

    def pair_body(s, carry):
        t0 = 2 * s
        t1 = 2 * s + 1
        cp0 = fire(t0, idxa0, idxb0, bufa0, bufb0, sa0, sb0)
        cp1 = fire(t1, idxa1, idxb1, bufa1, bufb1, sa1, sb1)
        cp0[0].wait()
        cp0[1].wait()
        add_store(t0, bufa0, bufb0)
        cp1[0].wait()
        cp1[1].wait()
        add_store(t1, bufa1, bufb1)
        return carry

    lax.fori_loop(0, cnt // 2, pair_body, 0)

    @pl.when(cnt % 2 == 1)
    def _tail():
        t = cnt - 1
        cp = fire(t, idxa0, idxb0, bufa0, bufb0, sa0, sb0)
        cp[0].wait()
        cp[1].wait()
        add_store(t, bufa0, bufb0)


@functools.cache
def _make_gather_add():
    return functools.partial(
        pl.kernel,
        out_type=jax.ShapeDtypeStruct((N_PAIRS, F), jnp.float32),
        mesh=plsc.VectorSubcoreMesh(core_axis_name="c", subcore_axis_name="s"),
        scratch_types=[
            pltpu.VMEM((_CHUNK,), jnp.int32),
            pltpu.VMEM((_CHUNK,), jnp.int32),
            pltpu.VMEM((_CHUNK,), jnp.int32),
            pltpu.VMEM((_CHUNK,), jnp.int32),
            pltpu.VMEM((_CHUNK, F), jnp.float32),
            pltpu.VMEM((_CHUNK, F), jnp.float32),
            pltpu.VMEM((_CHUNK, F), jnp.float32),
            pltpu.VMEM((_CHUNK, F), jnp.float32),
            pltpu.SemaphoreType.DMA,
            pltpu.SemaphoreType.DMA,
            pltpu.SemaphoreType.DMA,
            pltpu.SemaphoreType.DMA,
        ],
    )(_gather_add_body)


_MLP_BLK = 1280
_MLP_NB = N_PAIRS // _MLP_BLK


def _mlp_body(w_ref, g_ref, w1c_ref, b1_ref, w2_ref, b2_ref, out_ref):
    c = jnp.dot(w_ref[...].astype(jnp.bfloat16), w1c_ref[...].astype(jnp.bfloat16),
                preferred_element_type=jnp.float32)
    y = c + g_ref[...] + b1_ref[...]
    h = y * jax.nn.sigmoid(y)
    s = lax.dot_general(w2_ref[...].astype(jnp.bfloat16), h.astype(jnp.bfloat16),
                        dimension_numbers=(((1,), (1,)), ((), ())),
                        preferred_element_type=jnp.float32)
    out_ref[0, 0, :] = s[0] + b2_ref[0]


def _mlp(w_ij, g, w1c, b1_row, w2_col, b2):
    out = pl.pallas_call(
        _mlp_body,
        grid=(_MLP_NB,),
        in_specs=[
            pl.BlockSpec((_MLP_BLK, F), lambda i: (i, 0)),
            pl.BlockSpec((_MLP_BLK, F), lambda i: (i, 0)),
            pl.BlockSpec((F, F), lambda i: (0, 0)),
            pl.BlockSpec((1, F), lambda i: (0, 0)),
            pl.BlockSpec((1, F), lambda i: (0, 0)),
            pl.BlockSpec(memory_space=pltpu.SMEM),
        ],
        out_specs=pl.BlockSpec((1, 1, _MLP_BLK), lambda i: (i, 0, 0)),
        out_shape=jax.ShapeDtypeStruct((_MLP_NB, 1, _MLP_BLK), jnp.float32),
    )(w_ij, g, w1c, b1_row, w2_col, b2)
    return out.reshape(N_PAIRS)


def kernel(x, w_ij, idx_i, idx_j, W1, b1, W2, b2):
    w1a = W1[:F]
    w1b = W1[F:2 * F]
    w1c = W1[2 * F:]
    xa, xb = _premul(x, w1a, w1b)
    g = _make_gather_add()(xa, xb, idx_i.astype(jnp.int32), idx_j.astype(jnp.int32))
    return _mlp(w_ij, g, w1c, b1.reshape(1, F), W2.reshape(1, F), b2)
